# baseline (speedup 1.0000x reference)
import numpy as np

import jax
import jax.numpy as jnp
from jax.experimental import pallas as pl
from jax.experimental.pallas import tpu as pltpu

_BI = 128


def _lenet_body(x_ref, w1_ref, b1_ref, w2_ref, b2_ref,
                wfa_ref, bf1_ref, wf2_ref, bf2_ref, wf3_ref, bf3_ref,
                o_ref):
    f32 = jnp.float32
    m1 = _BI * 32

    xp = jnp.concatenate([x_ref[...], jnp.zeros((8, 256), f32)], axis=0)
    acc = jnp.dot(xp[0:m1, :], w1_ref[0], preferred_element_type=f32)
    for i in range(1, 5):
        acc = acc + jnp.dot(xp[i:i + m1, :], w1_ref[i],
                            preferred_element_type=f32)
    y = jnp.maximum(acc + b1_ref[...], 0.0)

    y = y.reshape(m1 // 2, 2, 256)
    v = jnp.maximum(y[:, 0], y[:, 1])
    vs = jnp.concatenate([v[:, 8:], jnp.zeros((m1 // 2, 8), f32)], axis=1)
    p1 = jnp.maximum(v, vs)

    m2 = m1 // 2
    p1p = jnp.concatenate([p1, jnp.zeros((8, 256), f32)], axis=0)
    acc2 = jnp.dot(p1p[0:m2, :], w2_ref[0], preferred_element_type=f32)
    for i in range(1, 5):
        acc2 = acc2 + jnp.dot(p1p[i:i + m2, :], w2_ref[i],
                              preferred_element_type=f32)
    y2 = jnp.maximum(acc2 + b2_ref[...], 0.0)

    y2 = y2.reshape(m2 // 2, 2, 256)
    v2 = jnp.maximum(y2[:, 0], y2[:, 1])
    v2s = jnp.concatenate([v2[:, 16:], jnp.zeros((m2 // 2, 16), f32)], axis=1)
    p2 = jnp.maximum(v2, v2s)

    p2r = p2.reshape(_BI, 8, 256)
    h = jnp.dot(p2r[:, 0, :], wfa_ref[0], preferred_element_type=f32)
    for h4 in range(1, 5):
        h = h + jnp.dot(p2r[:, h4, :], wfa_ref[h4],
                        preferred_element_type=f32)
    h = jnp.maximum(h + bf1_ref[...], 0.0)

    h = jnp.dot(h, wf2_ref[...], preferred_element_type=f32) + bf2_ref[...]
    h = jnp.maximum(h, 0.0)
    o = jnp.dot(h, wf3_ref[...], preferred_element_type=f32) + bf3_ref[...]
    o_ref[...] = o


def _shift_eye(j):
    return np.eye(32, k=-j, dtype=np.float32)


def _fc1_onehot():
    m = np.zeros((5, 256, 640), np.float32)
    for h4 in range(5):
        for co in range(16):
            for s in range(5):
                m[h4, 32 * s + co, co * 40 + h4 * 8 + s] = 1.0
    return m

_FC1_MAP = _fc1_onehot()


def kernel(x, w1, b1, w2, b2, s1e, s1o, s2e, s2o,
           wf1, bf1, wf2, bf2, wf3, bf3):
    n = x.shape[0]
    npad = -(-n // _BI) * _BI

    xw = jnp.transpose(x, (0, 2, 3, 1))
    xw = jnp.pad(xw, ((0, npad - n), (0, 0), (0, 0), (0, 5)))
    xw = xw.reshape(npad * 32, 256)

    w1r = w1.reshape(5, 5, 8, 8)
    w1s = jnp.stack([
        sum(jnp.kron(jnp.asarray(_shift_eye(j)), w1r[i, j]) for j in range(5))
        for i in range(5)])
    b1l = jnp.tile(b1, (1, 32))

    w2r = w2.reshape(5, 5, 8, 16)
    w2p = jnp.pad(w2r, ((0, 0), (0, 0), (0, 8), (0, 0)))
    w2s = jnp.stack([
        sum(jnp.kron(jnp.asarray(np.eye(16, k=-j, dtype=np.float32)),
                     w2p[i, j]) for j in range(5))
        for i in range(5)])
    b2l = jnp.tile(b2, (1, 16))

    wfa = jnp.einsum('hpf,fo->hpo', jnp.asarray(_FC1_MAP), wf1)

    out = pl.pallas_call(
        _lenet_body,
        out_shape=jax.ShapeDtypeStruct((npad, 128), jnp.float32),
        grid=(npad // _BI,),
        in_specs=[
            pl.BlockSpec((_BI * 32, 256), lambda b: (b, 0)),
            pl.BlockSpec((5, 256, 256), lambda b: (0, 0, 0)),
            pl.BlockSpec((1, 256), lambda b: (0, 0)),
            pl.BlockSpec((5, 256, 256), lambda b: (0, 0, 0)),
            pl.BlockSpec((1, 256), lambda b: (0, 0)),
            pl.BlockSpec((5, 256, 128), lambda b: (0, 0, 0)),
            pl.BlockSpec((1, 128), lambda b: (0, 0)),
            pl.BlockSpec((128, 128), lambda b: (0, 0)),
            pl.BlockSpec((1, 128), lambda b: (0, 0)),
            pl.BlockSpec((128, 128), lambda b: (0, 0)),
            pl.BlockSpec((1, 128), lambda b: (0, 0)),
        ],
        out_specs=pl.BlockSpec((_BI, 128), lambda b: (b, 0)),
        compiler_params=pltpu.CompilerParams(
            dimension_semantics=("parallel",)),
    )(xw, w1s, b1l, w2s, b2l, wfa, bf1, wf2, bf2, wf3, bf3)

    return jax.lax.slice(out, (0, 0), (n, 10))


# speedup vs baseline: 3.1612x; 3.1612x over previous
import numpy as np

import jax
import jax.numpy as jnp
from jax.experimental import pallas as pl
from jax.experimental.pallas import tpu as pltpu

_BI = 128


def _lenet_body(x_ref, sc_ref, w1_ref, b1_ref, w2_ref, b2_ref,
                wfa_ref, bf1_ref, wf2_ref, bf2_ref, wf3_ref, bf3_ref,
                o_ref):
    f32 = jnp.float32
    m1 = _BI * 32

    xr = x_ref[...].reshape(_BI, 3, 32, 32)
    xw = jnp.dot(xr[:, 0].reshape(m1, 32), sc_ref[0],
                 preferred_element_type=f32)
    for ci in range(1, 3):
        xw = xw + jnp.dot(xr[:, ci].reshape(m1, 32), sc_ref[ci],
                          preferred_element_type=f32)

    xp = jnp.concatenate([xw, jnp.zeros((8, 128), f32)], axis=0)
    acc = jnp.dot(xp[0:m1, :], w1_ref[0], preferred_element_type=f32)
    for i in range(1, 5):
        acc = acc + jnp.dot(xp[i:i + m1, :], w1_ref[i],
                            preferred_element_type=f32)
    y = jnp.maximum(acc + b1_ref[...], 0.0)

    y = y.reshape(m1 // 2, 2, 256)
    v = jnp.maximum(y[:, 0], y[:, 1])
    p1 = jnp.maximum(v[:, 0:128], v[:, 128:256])

    m2 = m1 // 2
    p1p = jnp.concatenate([p1, jnp.zeros((8, 128), f32)], axis=0)
    acc2 = jnp.dot(p1p[0:m2, :], w2_ref[0], preferred_element_type=f32)
    for i in range(1, 5):
        acc2 = acc2 + jnp.dot(p1p[i:i + m2, :], w2_ref[i],
                              preferred_element_type=f32)
    y2 = jnp.maximum(acc2 + b2_ref[...], 0.0)

    y2 = y2.reshape(m2 // 2, 2, 256)
    v2 = jnp.maximum(y2[:, 0], y2[:, 1])
    p2 = jnp.maximum(v2[:, 0:128], v2[:, 128:256])

    p2r = p2.reshape(_BI, 8, 128)
    h = jnp.dot(p2r[:, 0, :], wfa_ref[0], preferred_element_type=f32)
    for h4 in range(1, 5):
        h = h + jnp.dot(p2r[:, h4, :], wfa_ref[h4],
                        preferred_element_type=f32)
    h = jnp.maximum(h + bf1_ref[...], 0.0)

    h = jnp.dot(h, wf2_ref[...], preferred_element_type=f32) + bf2_ref[...]
    h = jnp.maximum(h, 0.0)
    o = jnp.dot(h, wf3_ref[...], preferred_element_type=f32) + bf3_ref[...]
    o_ref[...] = o


def _a1_maps():
    a = np.zeros((5, 32, 32), np.float32)
    for j in range(5):
        for m in range(32):
            w_out = 2 * (m % 16) + (m // 16)
            if w_out + j < 32:
                a[j, w_out + j, m] = 1.0
    return a


def _a2_maps():
    a = np.zeros((5, 16, 16), np.float32)
    for j in range(5):
        for mm in range(16):
            w_out = 2 * (mm % 8) + (mm // 8)
            if w_out + j < 16:
                a[j, w_out + j, mm] = 1.0
    return a


def _fc1_onehot():
    m = np.zeros((5, 128, 640), np.float32)
    for h4 in range(5):
        for co in range(16):
            for s in range(5):
                m[h4, 16 * s + co, co * 40 + h4 * 8 + s] = 1.0
    return m


def _spread_onehot():
    m = np.zeros((3, 32, 128), np.float32)
    for ci in range(3):
        for w in range(32):
            m[ci, w, 4 * w + ci] = 1.0
    return m

_A1 = _a1_maps()
_A2 = _a2_maps()
_FC1_MAP = _fc1_onehot()
_SPREAD = _spread_onehot()


def kernel(x, w1, b1, w2, b2, s1e, s1o, s2e, s2o,
           wf1, bf1, wf2, bf2, wf3, bf3):
    n = x.shape[0]
    npad = -(-n // _BI) * _BI

    xz = jnp.pad(x, ((0, npad - n), (0, 0), (0, 0), (0, 0)))
    xz = xz.reshape(npad * 96, 32)
    scm = jnp.asarray(_SPREAD)

    w1c = w1.reshape(5, 5, 8, 8)[:, :, :4, :]
    w1s = jnp.stack([
        sum(jnp.kron(jnp.asarray(_A1[j]), w1c[i, j]) for j in range(5))
        for i in range(5)])
    b1l = jnp.tile(b1, (1, 32))

    w2r = w2.reshape(5, 5, 8, 16)
    w2s = jnp.stack([
        sum(jnp.kron(jnp.asarray(_A2[j]), w2r[i, j]) for j in range(5))
        for i in range(5)])
    b2l = jnp.tile(b2, (1, 16))

    wfa = jnp.einsum('hpf,fo->hpo', jnp.asarray(_FC1_MAP), wf1)

    out = pl.pallas_call(
        _lenet_body,
        out_shape=jax.ShapeDtypeStruct((npad, 128), jnp.float32),
        grid=(npad // _BI,),
        in_specs=[
            pl.BlockSpec((_BI * 96, 32), lambda b: (b, 0)),
            pl.BlockSpec((3, 32, 128), lambda b: (0, 0, 0)),
            pl.BlockSpec((5, 128, 256), lambda b: (0, 0, 0)),
            pl.BlockSpec((1, 256), lambda b: (0, 0)),
            pl.BlockSpec((5, 128, 256), lambda b: (0, 0, 0)),
            pl.BlockSpec((1, 256), lambda b: (0, 0)),
            pl.BlockSpec((5, 128, 128), lambda b: (0, 0, 0)),
            pl.BlockSpec((1, 128), lambda b: (0, 0)),
            pl.BlockSpec((128, 128), lambda b: (0, 0)),
            pl.BlockSpec((1, 128), lambda b: (0, 0)),
            pl.BlockSpec((128, 128), lambda b: (0, 0)),
            pl.BlockSpec((1, 128), lambda b: (0, 0)),
        ],
        out_specs=pl.BlockSpec((_BI, 128), lambda b: (b, 0)),
        compiler_params=pltpu.CompilerParams(
            dimension_semantics=("parallel",)),
    )(xz, scm, w1s, b1l, w2s, b2l, wfa, bf1, wf2, bf2, wf3, bf3)

    return jax.lax.slice(out, (0, 0), (n, 10))


# speedup vs baseline: 4.0453x; 1.2797x over previous
import numpy as np

import jax
import jax.numpy as jnp
from jax.experimental import pallas as pl
from jax.experimental.pallas import tpu as pltpu

_BI = 128


def _lenet_body(x_ref, sc_ref, w1_ref, b1_ref, w2_ref, b2_ref,
                wfa_ref, bf1_ref, wf2_ref, bf2_ref, wf3_ref, bf3_ref,
                o_ref):
    f32 = jnp.float32
    m1 = _BI * 32

    xr = x_ref[...].reshape(_BI, 3, 32, 32)
    xw = jnp.dot(xr[:, 0].reshape(m1, 32), sc_ref[0],
                 preferred_element_type=f32)
    for ci in range(1, 3):
        xw = xw + jnp.dot(xr[:, ci].reshape(m1, 32), sc_ref[ci],
                          preferred_element_type=f32)

    xp = jnp.concatenate([xw, jnp.zeros((8, 128), f32)], axis=0)
    acc = None
    for k in range(3):
        lhs = jnp.concatenate(
            [xp[2 * k:2 * k + m1, :], xp[2 * k + 1:2 * k + 1 + m1, :]],
            axis=1)
        d = jnp.dot(lhs, w1_ref[k], preferred_element_type=f32)
        acc = d if acc is None else acc + d
    y = jnp.maximum(acc + b1_ref[...], 0.0)

    y = y.reshape(m1 // 2, 2, 256)
    v = jnp.maximum(y[:, 0], y[:, 1])
    p1 = jnp.maximum(v[:, 0:128], v[:, 128:256])

    m2 = m1 // 2
    p1p = jnp.concatenate([p1, jnp.zeros((8, 128), f32)], axis=0)
    acc2 = None
    for k in range(3):
        lhs = jnp.concatenate(
            [p1p[2 * k:2 * k + m2, :], p1p[2 * k + 1:2 * k + 1 + m2, :]],
            axis=1)
        d = jnp.dot(lhs, w2_ref[k], preferred_element_type=f32)
        acc2 = d if acc2 is None else acc2 + d
    y2 = jnp.maximum(acc2 + b2_ref[...], 0.0)

    y2 = y2.reshape(m2 // 2, 2, 256)
    v2 = jnp.maximum(y2[:, 0], y2[:, 1])
    p2 = jnp.maximum(v2[:, 0:128], v2[:, 128:256])

    p2r = p2.reshape(_BI, 8, 128)
    h = jnp.dot(p2r[:, 0, :], wfa_ref[0], preferred_element_type=f32)
    for h4 in range(1, 5):
        h = h + jnp.dot(p2r[:, h4, :], wfa_ref[h4],
                        preferred_element_type=f32)
    h = jnp.maximum(h + bf1_ref[...], 0.0)

    h = jnp.dot(h, wf2_ref[...], preferred_element_type=f32) + bf2_ref[...]
    h = jnp.maximum(h, 0.0)
    o = jnp.dot(h, wf3_ref[...], preferred_element_type=f32) + bf3_ref[...]
    o_ref[...] = o


def _a1_maps():
    a = np.zeros((5, 32, 32), np.float32)
    for j in range(5):
        for m in range(32):
            w_out = 2 * (m % 16) + (m // 16)
            if w_out + j < 32:
                a[j, w_out + j, m] = 1.0
    return a


def _a2_maps():
    a = np.zeros((5, 16, 16), np.float32)
    for j in range(5):
        for mm in range(16):
            w_out = 2 * (mm % 8) + (mm // 8)
            if w_out + j < 16:
                a[j, w_out + j, mm] = 1.0
    return a


def _fc1_onehot():
    m = np.zeros((5, 128, 640), np.float32)
    for h4 in range(5):
        for co in range(16):
            for s in range(5):
                m[h4, 16 * s + co, co * 40 + h4 * 8 + s] = 1.0
    return m


def _spread_onehot():
    m = np.zeros((3, 32, 128), np.float32)
    for ci in range(3):
        for w in range(32):
            m[ci, w, 4 * w + ci] = 1.0
    return m

_A1 = _a1_maps()
_A2 = _a2_maps()
_FC1_MAP = _fc1_onehot()
_SPREAD = _spread_onehot()


def kernel(x, w1, b1, w2, b2, s1e, s1o, s2e, s2o,
           wf1, bf1, wf2, bf2, wf3, bf3):
    n = x.shape[0]
    npad = -(-n // _BI) * _BI

    xz = jnp.pad(x, ((0, npad - n), (0, 0), (0, 0), (0, 0)))
    xz = xz.reshape(npad * 96, 32)
    scm = jnp.asarray(_SPREAD)

    w1c = w1.reshape(5, 5, 8, 8)[:, :, :4, :]
    w1s = jnp.stack([
        sum(jnp.kron(jnp.asarray(_A1[j]), w1c[i, j]) for j in range(5))
        for i in range(5)])
    w1s = jnp.concatenate([w1s, jnp.zeros((1, 128, 256), jnp.float32)])
    w1s = w1s.reshape(3, 256, 256)
    b1l = jnp.tile(b1, (1, 32))

    w2r = w2.reshape(5, 5, 8, 16)
    w2s = jnp.stack([
        sum(jnp.kron(jnp.asarray(_A2[j]), w2r[i, j]) for j in range(5))
        for i in range(5)])
    w2s = jnp.concatenate([w2s, jnp.zeros((1, 128, 256), jnp.float32)])
    w2s = w2s.reshape(3, 256, 256)
    b2l = jnp.tile(b2, (1, 16))

    wfa = jnp.einsum('hpf,fo->hpo', jnp.asarray(_FC1_MAP), wf1)

    out = pl.pallas_call(
        _lenet_body,
        out_shape=jax.ShapeDtypeStruct((npad, 128), jnp.float32),
        grid=(npad // _BI,),
        in_specs=[
            pl.BlockSpec((_BI * 96, 32), lambda b: (b, 0)),
            pl.BlockSpec((3, 32, 128), lambda b: (0, 0, 0)),
            pl.BlockSpec((3, 256, 256), lambda b: (0, 0, 0)),
            pl.BlockSpec((1, 256), lambda b: (0, 0)),
            pl.BlockSpec((3, 256, 256), lambda b: (0, 0, 0)),
            pl.BlockSpec((1, 256), lambda b: (0, 0)),
            pl.BlockSpec((5, 128, 128), lambda b: (0, 0, 0)),
            pl.BlockSpec((1, 128), lambda b: (0, 0)),
            pl.BlockSpec((128, 128), lambda b: (0, 0)),
            pl.BlockSpec((1, 128), lambda b: (0, 0)),
            pl.BlockSpec((128, 128), lambda b: (0, 0)),
            pl.BlockSpec((1, 128), lambda b: (0, 0)),
        ],
        out_specs=pl.BlockSpec((_BI, 128), lambda b: (b, 0)),
        compiler_params=pltpu.CompilerParams(
            dimension_semantics=("parallel",)),
    )(xz, scm, w1s, b1l, w2s, b2l, wfa, bf1, wf2, bf2, wf3, bf3)

    return jax.lax.slice(out, (0, 0), (n, 10))


# speedup vs baseline: 4.1906x; 1.0359x over previous
import numpy as np

import jax
import jax.numpy as jnp
from jax.experimental import pallas as pl
from jax.experimental.pallas import tpu as pltpu

_BI = 256


def _lenet_body(x_ref, sc_ref, w1_ref, b1_ref, w2_ref, b2_ref,
                wfa_ref, bf1_ref, wf2_ref, bf2_ref, wf3_ref, bf3_ref,
                o_ref):
    f32 = jnp.float32
    m1 = _BI * 32

    xr = x_ref[...].reshape(_BI, 3, 32, 32)
    xw = jnp.dot(xr[:, 0].reshape(m1, 32), sc_ref[0],
                 preferred_element_type=f32)
    for ci in range(1, 3):
        xw = xw + jnp.dot(xr[:, ci].reshape(m1, 32), sc_ref[ci],
                          preferred_element_type=f32)

    xp = jnp.concatenate([xw, jnp.zeros((8, 128), f32)], axis=0)
    acc = None
    for k in range(3):
        lhs = jnp.concatenate(
            [xp[2 * k:2 * k + m1, :], xp[2 * k + 1:2 * k + 1 + m1, :]],
            axis=1)
        d = jnp.dot(lhs, w1_ref[k], preferred_element_type=f32)
        acc = d if acc is None else acc + d
    y = jnp.maximum(acc + b1_ref[...], 0.0)

    y = y.reshape(m1 // 2, 2, 256)
    v = jnp.maximum(y[:, 0], y[:, 1])
    p1 = jnp.maximum(v[:, 0:128], v[:, 128:256])

    m2 = m1 // 2
    p1p = jnp.concatenate([p1, jnp.zeros((8, 128), f32)], axis=0)
    acc2 = None
    for k in range(3):
        lhs = jnp.concatenate(
            [p1p[2 * k:2 * k + m2, :], p1p[2 * k + 1:2 * k + 1 + m2, :]],
            axis=1)
        d = jnp.dot(lhs, w2_ref[k], preferred_element_type=f32)
        acc2 = d if acc2 is None else acc2 + d
    y2 = jnp.maximum(acc2 + b2_ref[...], 0.0)

    y2 = y2.reshape(m2 // 2, 2, 256)
    v2 = jnp.maximum(y2[:, 0], y2[:, 1])
    p2 = jnp.maximum(v2[:, 0:128], v2[:, 128:256])

    p2r = p2.reshape(_BI, 8, 128)
    h = jnp.dot(p2r[:, 0, :], wfa_ref[0], preferred_element_type=f32)
    for h4 in range(1, 5):
        h = h + jnp.dot(p2r[:, h4, :], wfa_ref[h4],
                        preferred_element_type=f32)
    h = jnp.maximum(h + bf1_ref[...], 0.0)

    h = jnp.dot(h, wf2_ref[...], preferred_element_type=f32) + bf2_ref[...]
    h = jnp.maximum(h, 0.0)
    o = jnp.dot(h, wf3_ref[...], preferred_element_type=f32) + bf3_ref[...]
    o_ref[...] = o


def _a1_maps():
    a = np.zeros((5, 32, 32), np.float32)
    for j in range(5):
        for m in range(32):
            w_out = 2 * (m % 16) + (m // 16)
            if w_out + j < 32:
                a[j, w_out + j, m] = 1.0
    return a


def _a2_maps():
    a = np.zeros((5, 16, 16), np.float32)
    for j in range(5):
        for mm in range(16):
            w_out = 2 * (mm % 8) + (mm // 8)
            if w_out + j < 16:
                a[j, w_out + j, mm] = 1.0
    return a


def _fc1_onehot():
    m = np.zeros((5, 128, 640), np.float32)
    for h4 in range(5):
        for co in range(16):
            for s in range(5):
                m[h4, 16 * s + co, co * 40 + h4 * 8 + s] = 1.0
    return m


def _spread_onehot():
    m = np.zeros((3, 32, 128), np.float32)
    for ci in range(3):
        for w in range(32):
            m[ci, w, 4 * w + ci] = 1.0
    return m

_A1 = _a1_maps()
_A2 = _a2_maps()
_FC1_MAP = _fc1_onehot()
_SPREAD = _spread_onehot()


def kernel(x, w1, b1, w2, b2, s1e, s1o, s2e, s2o,
           wf1, bf1, wf2, bf2, wf3, bf3):
    n = x.shape[0]
    npad = -(-n // _BI) * _BI

    xz = jnp.pad(x, ((0, npad - n), (0, 0), (0, 0), (0, 0)))
    xz = xz.reshape(npad * 96, 32)
    scm = jnp.asarray(_SPREAD)

    w1c = w1.reshape(5, 5, 8, 8)[:, :, :4, :]
    w1s = jnp.stack([
        sum(jnp.kron(jnp.asarray(_A1[j]), w1c[i, j]) for j in range(5))
        for i in range(5)])
    w1s = jnp.concatenate([w1s, jnp.zeros((1, 128, 256), jnp.float32)])
    w1s = w1s.reshape(3, 256, 256)
    b1l = jnp.tile(b1, (1, 32))

    w2r = w2.reshape(5, 5, 8, 16)
    w2s = jnp.stack([
        sum(jnp.kron(jnp.asarray(_A2[j]), w2r[i, j]) for j in range(5))
        for i in range(5)])
    w2s = jnp.concatenate([w2s, jnp.zeros((1, 128, 256), jnp.float32)])
    w2s = w2s.reshape(3, 256, 256)
    b2l = jnp.tile(b2, (1, 16))

    wfa = jnp.einsum('hpf,fo->hpo', jnp.asarray(_FC1_MAP), wf1)

    out = pl.pallas_call(
        _lenet_body,
        out_shape=jax.ShapeDtypeStruct((npad, 128), jnp.float32),
        grid=(npad // _BI,),
        in_specs=[
            pl.BlockSpec((_BI * 96, 32), lambda b: (b, 0)),
            pl.BlockSpec((3, 32, 128), lambda b: (0, 0, 0)),
            pl.BlockSpec((3, 256, 256), lambda b: (0, 0, 0)),
            pl.BlockSpec((1, 256), lambda b: (0, 0)),
            pl.BlockSpec((3, 256, 256), lambda b: (0, 0, 0)),
            pl.BlockSpec((1, 256), lambda b: (0, 0)),
            pl.BlockSpec((5, 128, 128), lambda b: (0, 0, 0)),
            pl.BlockSpec((1, 128), lambda b: (0, 0)),
            pl.BlockSpec((128, 128), lambda b: (0, 0)),
            pl.BlockSpec((1, 128), lambda b: (0, 0)),
            pl.BlockSpec((128, 128), lambda b: (0, 0)),
            pl.BlockSpec((1, 128), lambda b: (0, 0)),
        ],
        out_specs=pl.BlockSpec((_BI, 128), lambda b: (b, 0)),
        compiler_params=pltpu.CompilerParams(
            dimension_semantics=("parallel",)),
    )(xz, scm, w1s, b1l, w2s, b2l, wfa, bf1, wf2, bf2, wf3, bf3)

    return jax.lax.slice(out, (0, 0), (n, 10))
